# Initial kernel scaffold; baseline (speedup 1.0000x reference)
import jax
import jax.numpy as jnp
from jax.experimental import pallas as pl


def kernel(x, y):
    raise NotImplementedError("write your pallas kernel here")


# speedup vs baseline: 48.3138x; 48.3138x over previous
import functools

import jax
import jax.numpy as jnp
import numpy as np
from jax import lax
from jax.experimental import pallas as pl
from jax.experimental.pallas import tpu as pltpu
from jax.experimental.pallas import tpu_sc as plsc

B = 128
N = 100000
L = 16
NW = 32
RPW = B // NW
CHUNK = 20000
NCHUNK = N // CHUNK
NVEC = CHUNK // L
UNROLL = 10
NEG = np.float32(-np.inf)


def _fix_label(buf, yv, c_base, s_y, iota):
    p = yv - c_base
    inchunk = (p >= 0) & (p < CHUNK)
    pc = jnp.clip(p, 0, CHUNK - 1)
    val = plsc.load_gather(buf, [pc])
    s_y = jnp.where(inchunk, val, s_y)
    zmask = inchunk & (iota == 0)
    plsc.store_scatter(buf, [pc], jnp.zeros((L,), jnp.float32), mask=zmask)
    return s_y


def _insert_chunk(buf, ts):
    def body(k, ts):
        t0, t1, t2, t3, t4 = ts
        base = k * (UNROLL * L)
        for u in range(UNROLL):
            v = buf[pl.ds(base + u * L, L)]
            m = jnp.maximum(t0, v); v = jnp.minimum(t0, v); t0 = m
            m = jnp.maximum(t1, v); v = jnp.minimum(t1, v); t1 = m
            m = jnp.maximum(t2, v); v = jnp.minimum(t2, v); t2 = m
            m = jnp.maximum(t3, v); v = jnp.minimum(t3, v); t3 = m
            t4 = jnp.maximum(t4, v)
        return (t0, t1, t2, t3, t4)

    return lax.fori_loop(0, NVEC // UNROLL, body, ts)


def _fifth(ts, iota):
    t0, t1, t2, t3, t4 = ts
    g = NEG
    for _ in range(5):
        sv, sl = plsc.sort_key_val(t0, iota, descending=True)
        g = sv[0]
        lanemask = iota == sl[0]
        t0 = jnp.where(lanemask, t1, t0)
        t1 = jnp.where(lanemask, t2, t1)
        t2 = jnp.where(lanemask, t3, t2)
        t3 = jnp.where(lanemask, t4, t3)
        t4 = jnp.where(lanemask, NEG, t4)
    return g


@functools.partial(
    pl.kernel,
    mesh=plsc.VectorSubcoreMesh(core_axis_name="c", subcore_axis_name="s"),
    out_type=jax.ShapeDtypeStruct((NW, L), jnp.float32),
    compiler_params=pltpu.CompilerParams(needs_layout_passes=False),
    scratch_types=[
        pltpu.VMEM((CHUNK,), jnp.float32),
        pltpu.VMEM((CHUNK,), jnp.float32),
        pltpu.VMEM((B,), jnp.float32),
        pltpu.VMEM((L,), jnp.float32),
        pltpu.SemaphoreType.DMA,
        pltpu.SemaphoreType.DMA,
    ],
)
def _sc_topk(xf_hbm, y_hbm, out_hbm, buf0, buf1, ybuf, outbuf, sem0, sem1):
    cid = lax.axis_index("c")
    sid = lax.axis_index("s")
    wid = sid * 2 + cid
    row0 = wid * RPW
    iota = lax.iota(jnp.int32, L)

    pltpu.sync_copy(y_hbm, ybuf)

    bufs = (buf0, buf1)
    sems = (sem0, sem1)
    NSTEP = RPW * NCHUNK

    def src(s):
        j, c = divmod(s, NCHUNK)
        return xf_hbm.at[pl.ds((row0 + j) * N + c * CHUNK, CHUNK)]

    handles = [None] * NSTEP
    handles[0] = pltpu.async_copy(src(0), bufs[0], sems[0])
    outv = jnp.zeros((L,), jnp.float32)
    for j in range(RPW):
        yv = plsc.load_gather(
            ybuf, [jnp.full((L,), row0 + j, jnp.int32)]).astype(jnp.int32)
        s_y = jnp.zeros((L,), jnp.float32)
        ts = tuple(jnp.full((L,), NEG, jnp.float32) for _ in range(5))
        for c in range(NCHUNK):
            s = j * NCHUNK + c
            handles[s].wait()
            if s + 1 < NSTEP:
                handles[s + 1] = pltpu.async_copy(
                    src(s + 1), bufs[(s + 1) % 2], sems[(s + 1) % 2])
            buf = bufs[s % 2]
            s_y = _fix_label(buf, yv, c * CHUNK, s_y, iota)
            ts = _insert_chunk(buf, ts)
        s_topk = _fifth(ts, iota)
        outv = jnp.where(iota == j, s_topk, outv)
        outv = jnp.where(iota == 4 + j, s_y, outv)
    outbuf[...] = outv
    pltpu.sync_copy(outbuf, out_hbm.at[wid])


def _loss_body(tk_ref, sy_ref, o_ref):
    a = tk_ref[...]
    b = sy_ref[...]
    h = jnp.maximum(1.0 + a - b, 0.0)
    o_ref[...] = (jnp.sum(h) * np.float32(1.0 / (B * B)))[None, None]


_loss = pl.pallas_call(
    _loss_body,
    out_shape=jax.ShapeDtypeStruct((1, 1), jnp.float32),
)


def kernel(x, y):
    xf = x.reshape(-1)
    o = _sc_topk(xf, y.astype(jnp.float32))
    tk = o[:, 0:4].reshape(1, B)
    sy = o[:, 4:8].reshape(B, 1)
    return _loss(tk, sy).reshape(())
